# baseline (speedup 1.0000x reference)
import functools

import jax
import jax.numpy as jnp
from jax.experimental import pallas as pl
from jax.experimental.pallas import tpu as pltpu

B = 16


def _qproj_body(q_ref, wqn_ref, bqn_ref, bn_ref, wqe_ref, bqe_ref, be_ref,
                qn_ref, qe_ref):
    q = q_ref[...]
    qn_ref[...] = (jnp.dot(q, wqn_ref[...], preferred_element_type=jnp.float32)
                   + bqn_ref[...] + bn_ref[...])
    qe_ref[...] = (jnp.dot(q, wqe_ref[...], preferred_element_type=jnp.float32)
                   + bqe_ref[...] + be_ref[...])


def _fused_body(efeat_ref, eids_ref, ewcat_ref, ewv_ref, ebv_ref,
                nfeat_ref, nids_ref, nwcat_ref, nwv_ref, nbv_ref,
                eout_ref, nout_ref, *, echunk, nchunk):
    efeat_t = jnp.transpose(efeat_ref[...].astype(jnp.bfloat16))
    eids = eids_ref[...].reshape(1, echunk)
    eoh = (eids == jax.lax.broadcasted_iota(jnp.int32, (B, echunk), 0)
           ).astype(jnp.bfloat16)
    ea = jnp.concatenate([eoh, efeat_t], axis=0)
    ex = jax.lax.dot_general(
        ewcat_ref[...].astype(jnp.bfloat16), ea,
        (((0,), (0,)), ((), ())), preferred_element_type=jnp.float32)
    et = jnp.tanh(ex).astype(jnp.bfloat16)
    es = jax.lax.dot_general(
        ewv_ref[...].astype(jnp.bfloat16), et,
        (((0,), (0,)), ((), ())), preferred_element_type=jnp.float32)
    es = es + ebv_ref[...]
    em = jnp.max(es, axis=0, keepdims=True)
    ee = jnp.exp(es - em)
    eout_ref[...] = (ee / jnp.sum(ee, axis=0, keepdims=True)).reshape(
        echunk // 128, 128)

    nfeat_t = jnp.transpose(nfeat_ref[...].astype(jnp.bfloat16))
    nids = nids_ref[0]
    noh = (nids == jax.lax.broadcasted_iota(jnp.int32, (B, nchunk), 0)
           ).astype(jnp.bfloat16)
    na = jnp.concatenate([noh, nfeat_t], axis=0)
    nx = jax.lax.dot_general(
        nwcat_ref[...].astype(jnp.bfloat16), na,
        (((0,), (0,)), ((), ())), preferred_element_type=jnp.float32)
    nt = jnp.tanh(nx).astype(jnp.bfloat16)
    ns = jax.lax.dot_general(
        nwv_ref[...].astype(jnp.bfloat16), nt,
        (((0,), (0,)), ((), ())), preferred_element_type=jnp.float32)
    ns = ns + nbv_ref[...]
    nm = jnp.max(ns, axis=0, keepdims=True)
    ne = jnp.exp(ns - nm)
    nout_ref[...] = (ne / jnp.sum(ne, axis=0, keepdims=True))[None]


def kernel(question, node_feat, edge_feat, node_graph_ids, edge_graph_ids,
           Wqn, bqn, Wn, bn, wvn, bvn, Wqe, bqe, We, be, wve, bve):
    qn, qe = pl.pallas_call(
        _qproj_body,
        out_shape=(jax.ShapeDtypeStruct((B, Wqn.shape[1]), jnp.float32),
                   jax.ShapeDtypeStruct((B, Wqe.shape[1]), jnp.float32)),
    )(question, Wqn, bqn.reshape(1, -1), bn.reshape(1, -1),
      Wqe, bqe.reshape(1, -1), be.reshape(1, -1))

    ne_, de = edge_feat.shape
    nn, dn = node_feat.shape
    echunk, nchunk = 25600, 800
    grid = ne_ // echunk
    body = functools.partial(_fused_body, echunk=echunk, nchunk=nchunk)
    eids2 = edge_graph_ids.astype(jnp.int32).reshape(ne_ // 128, 128)
    nids3 = node_graph_ids.astype(jnp.int32).reshape(grid, 1, nchunk)
    eout, nout = pl.pallas_call(
        body,
        grid=(grid,),
        in_specs=[
            pl.BlockSpec((echunk, de), lambda i: (i, 0)),
            pl.BlockSpec((echunk // 128, 128), lambda i: (i, 0)),
            pl.BlockSpec((B + de, 128), lambda i: (0, 0)),
            pl.BlockSpec((128, 1), lambda i: (0, 0)),
            pl.BlockSpec((1, 1), lambda i: (0, 0)),
            pl.BlockSpec((nchunk, dn), lambda i: (i, 0)),
            pl.BlockSpec((1, 1, nchunk), lambda i: (i, 0, 0)),
            pl.BlockSpec((B + dn, 128), lambda i: (0, 0)),
            pl.BlockSpec((128, 1), lambda i: (0, 0)),
            pl.BlockSpec((1, 1), lambda i: (0, 0)),
        ],
        out_specs=(
            pl.BlockSpec((echunk // 128, 128), lambda i: (i, 0)),
            pl.BlockSpec((1, 1, nchunk), lambda i: (i, 0, 0)),
        ),
        out_shape=(
            jax.ShapeDtypeStruct((ne_ // 128, 128), jnp.float32),
            jax.ShapeDtypeStruct((grid, 1, nchunk), jnp.float32),
        ),
        compiler_params=pltpu.CompilerParams(
            dimension_semantics=("parallel",)),
    )(edge_feat, eids2, jnp.concatenate([qe, We]), wve, bve.reshape(1, 1),
      node_feat, nids3, jnp.concatenate([qn, Wn]), wvn, bvn.reshape(1, 1))
    return (nout.reshape(-1), eout.reshape(-1))
